# baseline (speedup 1.0000x reference)
import functools

import jax
import jax.numpy as jnp
from jax import lax
from jax.experimental import pallas as pl
from jax.experimental.pallas import tpu as pltpu
from jax.experimental.pallas import tpu_sc as plsc

B = 16384
C = 50
D = 32
VU = 1000000

NC = 2
NS = 16
NW = NC * NS
BPW = B // NW
CH = 32
NCHUNK = BPW // CH
TAG_SUB = 100
SUBS = CH * C // TAG_SUB
PR = CH * C
NWORD = CH * D
WSUB = NWORD // 128


def _sc_kernel(user_h, item_h, tag_h, tu_h, ti_h, tt_h, out_h,
               uraw_v, iraw_v, widx_u, widx_i, tidx_v,
               uw_v, iw_v, trows_v, out_v, sem):
    wid = lax.axis_index("s") * NC + lax.axis_index("c")
    iota = lax.iota(jnp.int32, 16)

    @pl.loop(0, NCHUNK)
    def _chunk(ch):
        b0 = pl.multiple_of(wid * BPW + ch * CH, CH)
        trow0 = pl.multiple_of(b0 * C // TAG_SUB, 16)

        pltpu.sync_copy(user_h.at[pl.ds(b0, CH)], uraw_v)
        pltpu.sync_copy(item_h.at[pl.ds(b0, CH)], iraw_v)
        pltpu.sync_copy(tag_h.at[pl.ds(trow0, SUBS)], tidx_v)

        ulo, uhi = uraw_v[pl.ds(0, 16)], uraw_v[pl.ds(16, 16)]
        ilo, ihi = iraw_v[pl.ds(0, 16)], iraw_v[pl.ds(16, 16)]
        for d in range(D):
            widx_u[pl.ds(d * CH, 16)] = ulo + d * VU
            widx_u[pl.ds(d * CH + 16, 16)] = uhi + d * VU
            widx_i[pl.ds(d * CH, 16)] = ilo + d * VU
            widx_i[pl.ds(d * CH + 16, 16)] = ihi + d * VU

        copies = []
        for j in range(WSUB):
            copies.append(
                pltpu.async_copy(tu_h.at[widx_u.at[pl.ds(j * 128, 128)]],
                                 uw_v.at[pl.ds(j * 128, 128)], sem))
            copies.append(
                pltpu.async_copy(ti_h.at[widx_i.at[pl.ds(j * 128, 128)]],
                                 iw_v.at[pl.ds(j * 128, 128)], sem))
        for j in range(SUBS):
            copies.append(
                pltpu.async_copy(tt_h.at[tidx_v.at[j]],
                                 trows_v.at[pl.ds(j * TAG_SUB, TAG_SUB)],
                                 sem))
        for cp in copies:
            cp.wait()

        @pl.loop(0, CH)
        def _row(b):
            dvec = iota * CH + b
            s_lo = (plsc.load_gather(uw_v, [dvec])
                    + plsc.load_gather(iw_v, [dvec]))
            s_hi = (plsc.load_gather(uw_v, [dvec + 16 * CH])
                    + plsc.load_gather(iw_v, [dvec + 16 * CH]))
            for g in range(4):
                nlc = 16 if g < 3 else C - 48
                acc = jnp.zeros((16,), jnp.float32)
                for lc in range(nlc):
                    p = b * C + g * 16 + lc
                    t_lo = trows_v[p, pl.ds(0, 16)]
                    t_hi = trows_v[p, pl.ds(16, 16)]
                    r = jnp.sum(s_lo * t_lo + s_hi * t_hi)
                    acc = jnp.where(iota == lc, r, acc)
                plsc.store_scatter(out_v, [iota + (b * C + g * 16)], acc,
                                   mask=iota < nlc)

        pltpu.sync_copy(out_v, out_h.at[pl.ds(b0 * C, PR)])


def kernel(user, item, tag, tag_type, table_u, table_i, table_t):
    del tag_type
    user = user.astype(jnp.int32)
    item = item.astype(jnp.int32)
    tag2 = tag.astype(jnp.int32).reshape(B * C // TAG_SUB, TAG_SUB)

    mesh = plsc.VectorSubcoreMesh(core_axis_name="c", subcore_axis_name="s")
    run = functools.partial(
        pl.kernel,
        out_type=jax.ShapeDtypeStruct((B * C,), jnp.float32),
        mesh=mesh,
        compiler_params=pltpu.CompilerParams(needs_layout_passes=False,
                                             use_tc_tiling_on_sc=False),
        scratch_types=[
            pltpu.VMEM((CH,), jnp.int32),
            pltpu.VMEM((CH,), jnp.int32),
            pltpu.VMEM((NWORD,), jnp.int32),
            pltpu.VMEM((NWORD,), jnp.int32),
            pltpu.VMEM((SUBS, TAG_SUB), jnp.int32),
            pltpu.VMEM((NWORD,), jnp.float32),
            pltpu.VMEM((NWORD,), jnp.float32),
            pltpu.VMEM((PR, D), jnp.float32),
            pltpu.VMEM((PR,), jnp.float32),
            pltpu.SemaphoreType.DMA,
        ],
    )(_sc_kernel)
    return run(user, item, tag2, table_u.T.reshape(-1), table_i.T.reshape(-1),
               table_t).reshape(B, C)


# speedup vs baseline: 19.8460x; 19.8460x over previous
import functools

import jax
import jax.numpy as jnp
from jax import lax
from jax.experimental import pallas as pl
from jax.experimental.pallas import tpu as pltpu
from jax.experimental.pallas import tpu_sc as plsc

B = 16384
C = 50
D = 32

NC = 2
NS = 16
NW = NC * NS
BPW = B // NW
CH = 32
NCHUNK = BPW // CH
TAG_SUB = 100
SUBS = CH * C // TAG_SUB
PR = CH * C


def _sc_kernel(s_h, tag_h, tt_h, out_h,
               srows_v, tidx_v, trows_v, out_v, sem):
    wid = lax.axis_index("s") * NC + lax.axis_index("c")
    iota = lax.iota(jnp.int32, 16)

    @pl.loop(0, NCHUNK)
    def _chunk(ch):
        b0 = pl.multiple_of(wid * BPW + ch * CH, CH)
        trow0 = pl.multiple_of(b0 * C // TAG_SUB, 16)

        pltpu.sync_copy(s_h.at[pl.ds(b0, CH)], srows_v)
        pltpu.sync_copy(tag_h.at[pl.ds(trow0, SUBS)], tidx_v)

        copies = []
        for j in range(SUBS):
            copies.append(
                pltpu.async_copy(tt_h.at[tidx_v.at[j]],
                                 trows_v.at[pl.ds(j * TAG_SUB, TAG_SUB)],
                                 sem))
        for cp in copies:
            cp.wait()

        @pl.loop(0, CH)
        def _row(b):
            s_lo = srows_v[b, pl.ds(0, 16)]
            s_hi = srows_v[b, pl.ds(16, 16)]
            for g in range(4):
                nlc = 16 if g < 3 else C - 48
                acc = jnp.zeros((16,), jnp.float32)
                for lc in range(nlc):
                    p = b * C + g * 16 + lc
                    t_lo = trows_v[p, pl.ds(0, 16)]
                    t_hi = trows_v[p, pl.ds(16, 16)]
                    r = jnp.sum(s_lo * t_lo + s_hi * t_hi)
                    acc = jnp.where(iota == lc, r, acc)
                plsc.store_scatter(out_v, [iota + (b * C + g * 16)], acc,
                                   mask=iota < nlc)

        pltpu.sync_copy(out_v, out_h.at[pl.ds(b0 * C, PR)])


def kernel(user, item, tag, tag_type, table_u, table_i, table_t):
    del tag_type
    user = user.astype(jnp.int32)
    item = item.astype(jnp.int32)
    tag2 = tag.astype(jnp.int32).reshape(B * C // TAG_SUB, TAG_SUB)
    s = jnp.take(table_u, user, axis=0) + jnp.take(table_i, item, axis=0)

    mesh = plsc.VectorSubcoreMesh(core_axis_name="c", subcore_axis_name="s")
    run = functools.partial(
        pl.kernel,
        out_type=jax.ShapeDtypeStruct((B * C,), jnp.float32),
        mesh=mesh,
        compiler_params=pltpu.CompilerParams(needs_layout_passes=False,
                                             use_tc_tiling_on_sc=False),
        scratch_types=[
            pltpu.VMEM((CH, D), jnp.float32),
            pltpu.VMEM((SUBS, TAG_SUB), jnp.int32),
            pltpu.VMEM((PR, D), jnp.float32),
            pltpu.VMEM((PR,), jnp.float32),
            pltpu.SemaphoreType.DMA,
        ],
    )(_sc_kernel)
    return run(s, tag2, table_t).reshape(B, C)


# speedup vs baseline: 25.4471x; 1.2822x over previous
import functools

import jax
import jax.numpy as jnp
from jax import lax
from jax.experimental import pallas as pl
from jax.experimental.pallas import tpu as pltpu
from jax.experimental.pallas import tpu_sc as plsc

B = 16384
C = 50
D = 32

NC = 2
NS = 16
NW = NC * NS
BPW = B // NW
CH = 16
NCHUNK = BPW // CH
TAG_SUB = 100
SUBS = CH * C // TAG_SUB
PR = CH * C
TROWS = BPW * C // TAG_SUB


def _sc_kernel(s_h, tag_h, tt_h, out_h,
               srows_v, tidx_v, trows_v, out_v, gsem0, gsem1, osem0, osem1):
    wid = lax.axis_index("s") * NC + lax.axis_index("c")
    iota = lax.iota(jnp.int32, 16)
    gsem = (gsem0, gsem1)
    osem = (osem0, osem1)
    wb0 = pl.multiple_of(wid * BPW, BPW)

    pltpu.sync_copy(s_h.at[pl.ds(wb0, BPW)], srows_v)
    pltpu.sync_copy(tag_h.at[pl.ds(pl.multiple_of(wid * TROWS, TROWS), TROWS)],
                    tidx_v)

    def fire(k, par):
        for j in range(SUBS):
            pltpu.async_copy(tt_h.at[tidx_v.at[k * SUBS + j]],
                             trows_v.at[par, pl.ds(j * TAG_SUB, TAG_SUB)],
                             gsem[par])

    fire(0, 0)

    @pl.loop(0, NCHUNK // 2)
    def _pair(cp):
        for par in (0, 1):
            k = cp * 2 + par

            @pl.when(k + 1 < NCHUNK)
            def _prefetch():
                fire(k + 1, 1 - par)

            pltpu.make_async_copy(tt_h.at[pl.ds(0, PR)],
                                  trows_v.at[par], gsem[par]).wait()

            @pl.when(k >= 2)
            def _reclaim():
                pltpu.make_async_copy(out_v.at[par],
                                      out_h.at[pl.ds(0, PR)],
                                      osem[par]).wait()

            @pl.loop(0, CH)
            def _row(b):
                sr = k * CH + b
                s_lo = srows_v[sr, pl.ds(0, 16)]
                s_hi = srows_v[sr, pl.ds(16, 16)]
                for g in range(4):
                    nlc = 16 if g < 3 else C - 48
                    acc = jnp.zeros((16,), jnp.float32)
                    for lc in range(nlc):
                        p = b * C + g * 16 + lc
                        t_lo = trows_v[par, p, pl.ds(0, 16)]
                        t_hi = trows_v[par, p, pl.ds(16, 16)]
                        r = jnp.sum(s_lo * t_lo + s_hi * t_hi)
                        acc = jnp.where(iota == lc, r, acc)
                    plsc.store_scatter(out_v.at[par],
                                       [iota + (b * C + g * 16)], acc,
                                       mask=iota < nlc)

            pltpu.async_copy(out_v.at[par],
                             out_h.at[pl.ds((wb0 + k * CH) * C, PR)],
                             osem[par])

    for par in (0, 1):
        pltpu.make_async_copy(out_v.at[par], out_h.at[pl.ds(0, PR)],
                              osem[par]).wait()


def kernel(user, item, tag, tag_type, table_u, table_i, table_t):
    del tag_type
    user = user.astype(jnp.int32)
    item = item.astype(jnp.int32)
    tag2 = tag.astype(jnp.int32).reshape(B * C // TAG_SUB, TAG_SUB)
    s = jnp.take(table_u, user, axis=0) + jnp.take(table_i, item, axis=0)

    mesh = plsc.VectorSubcoreMesh(core_axis_name="c", subcore_axis_name="s")
    run = functools.partial(
        pl.kernel,
        out_type=jax.ShapeDtypeStruct((B * C,), jnp.float32),
        mesh=mesh,
        compiler_params=pltpu.CompilerParams(needs_layout_passes=False,
                                             use_tc_tiling_on_sc=False),
        scratch_types=[
            pltpu.VMEM((BPW, D), jnp.float32),
            pltpu.VMEM((TROWS, TAG_SUB), jnp.int32),
            pltpu.VMEM((2, PR, D), jnp.float32),
            pltpu.VMEM((2, PR), jnp.float32),
            pltpu.SemaphoreType.DMA,
            pltpu.SemaphoreType.DMA,
            pltpu.SemaphoreType.DMA,
            pltpu.SemaphoreType.DMA,
        ],
    )(_sc_kernel)
    return run(s, tag2, table_t).reshape(B, C)
